# Initial kernel scaffold; baseline (speedup 1.0000x reference)
import jax
import jax.numpy as jnp
from jax.experimental import pallas as pl


def kernel(x, edge_index, edge_index_for_edge_prediction, W1, b1, W2, b2, ek_b, ek_s, Wf, bf, fk_b, fk_s, edk1_b, edk1_s, edk2_b, edk2_s, dk1_b, dk1_s, dk2_b, dk2_s):
    raise NotImplementedError("write your pallas kernel here")


# speedup vs baseline: 6.7174x; 6.7174x over previous
import functools

import numpy as np
import jax
import jax.numpy as jnp
from jax import lax
from jax.experimental import pallas as pl
from jax.experimental.pallas import tpu as pltpu
from jax.experimental.pallas import tpu_sc as plsc

F32 = jnp.float32

_H = np.float32(2.0) / np.float32(5.0)
_G = [np.float32(t) * _H - np.float32(1.0) for t in range(-3, 9)]

_NC, _NS = 2, 16
_NW = _NC * _NS
_NP = 10240


def _bases8(x):
    b = [((x >= _G[j]) & (x < _G[j + 1])).astype(x.dtype) for j in range(11)]
    for k in range(1, 4):
        b = [
            (x - _G[j]) / np.float32(_G[j + k] - _G[j]) * b[j]
            + (np.float32(_G[j + k + 1]) - x)
            / np.float32(_G[j + k + 1] - _G[j + 1]) * b[j + 1]
            for j in range(11 - k)
        ]
    return b


def _silu(x):
    return x * jax.nn.sigmoid(x)


def _kan(x, bT, sT):
    bs = jnp.concatenate(_bases8(x), axis=1)
    return (jnp.dot(_silu(x), bT, preferred_element_type=F32)
            + jnp.dot(bs, sT, preferred_element_type=F32))


def _kan_out1(x, b_row, s_rows):
    acc = jnp.sum(_silu(x) * b_row, axis=1, keepdims=True)
    bs = _bases8(x)
    for k in range(8):
        acc = acc + jnp.sum(bs[k] * s_rows[k:k + 1, :], axis=1, keepdims=True)
    return acc


def _dinv_of(cnt):
    return lax.rsqrt(cnt[0, :, 0:1] + cnt[1, :, 0:1] + 1.0)


_BN = 400
_BE = 1000


def _tc_scale_matmul(x, W, cnt):
    N, Fi = x.shape
    nb = N // _BN

    def body(x_ref, w_ref, c_ref, o_ref):
        dinv = _dinv_of(c_ref[...])
        o_ref[...] = jnp.dot(x_ref[...], w_ref[...],
                             preferred_element_type=F32) * dinv

    return pl.pallas_call(
        body,
        grid=(nb,),
        in_specs=[
            pl.BlockSpec((_BN, Fi), lambda i: (i, 0)),
            pl.BlockSpec((Fi, 128), lambda i: (0, 0)),
            pl.BlockSpec((2, _BN, 128), lambda i: (0, i, 0)),
        ],
        out_specs=pl.BlockSpec((_BN, 128), lambda i: (i, 0)),
        out_shape=jax.ShapeDtypeStruct((N, 128), F32),
    )(x, W, cnt)


def _tc_combine_matmul(parts, xws, cnt, b, W):
    N = xws.shape[0]
    nb = N // _BN

    def body(p_ref, x_ref, c_ref, b_ref, w_ref, o_ref):
        pv = p_ref[...]
        dinv = _dinv_of(c_ref[...])
        h = jax.nn.relu(dinv * (pv[0] + pv[1] + x_ref[...]) + b_ref[...])
        o_ref[...] = jnp.dot(h, w_ref[...], preferred_element_type=F32) * dinv

    return pl.pallas_call(
        body,
        grid=(nb,),
        in_specs=[
            pl.BlockSpec((2, _BN, 128), lambda i: (0, i, 0)),
            pl.BlockSpec((_BN, 128), lambda i: (i, 0)),
            pl.BlockSpec((2, _BN, 128), lambda i: (0, i, 0)),
            pl.BlockSpec((1, 128), lambda i: (0, 0)),
            pl.BlockSpec((128, 128), lambda i: (0, 0)),
        ],
        out_specs=pl.BlockSpec((_BN, 128), lambda i: (i, 0)),
        out_shape=jax.ShapeDtypeStruct((N, 128), F32),
    )(parts, xws, cnt, b, W)


def _tc_combine_kan_matmul(parts, xws, cnt, b, ekbT, eksT, Wf):
    N = xws.shape[0]
    nb = N // _BN

    def body(p_ref, x_ref, c_ref, b_ref, ekb_ref, eks_ref, wf_ref,
             z_ref, zp_ref, o_ref):
        pv = p_ref[...]
        dinv = _dinv_of(c_ref[...])
        h2 = jax.nn.relu(dinv * (pv[0] + pv[1] + x_ref[...]) + b_ref[...])
        z = _kan(h2, ekb_ref[...], eks_ref[...])
        z_ref[...] = z
        zp_ref[...] = jnp.concatenate([z, jnp.zeros_like(z)], axis=1)
        o_ref[...] = jnp.dot(z, wf_ref[...], preferred_element_type=F32) * dinv

    return pl.pallas_call(
        body,
        grid=(nb,),
        in_specs=[
            pl.BlockSpec((2, _BN, 128), lambda i: (0, i, 0)),
            pl.BlockSpec((_BN, 128), lambda i: (i, 0)),
            pl.BlockSpec((2, _BN, 128), lambda i: (0, i, 0)),
            pl.BlockSpec((1, 128), lambda i: (0, 0)),
            pl.BlockSpec((128, 64), lambda i: (0, 0)),
            pl.BlockSpec((1024, 64), lambda i: (0, 0)),
            pl.BlockSpec((64, 128), lambda i: (0, 0)),
        ],
        out_specs=[
            pl.BlockSpec((_BN, 64), lambda i: (i, 0)),
            pl.BlockSpec((_BN, 128), lambda i: (i, 0)),
            pl.BlockSpec((_BN, 128), lambda i: (i, 0)),
        ],
        out_shape=[
            jax.ShapeDtypeStruct((N, 64), F32),
            jax.ShapeDtypeStruct((N, 128), F32),
            jax.ShapeDtypeStruct((N, 128), F32),
        ],
    )(parts, xws, cnt, b, ekbT, eksT, Wf)


def _tc_combine_kan(parts, xws, cnt, b, fkbT, fksT):
    N = xws.shape[0]
    nb = N // _BN

    def body(p_ref, x_ref, c_ref, b_ref, fkb_ref, fks_ref, o_ref):
        pv = p_ref[...]
        dinv = _dinv_of(c_ref[...])
        h = jax.nn.relu(dinv * (pv[0] + pv[1] + x_ref[...]) + b_ref[...])
        o_ref[...] = _kan(h, fkb_ref[...], fks_ref[...])

    return pl.pallas_call(
        body,
        grid=(nb,),
        in_specs=[
            pl.BlockSpec((2, _BN, 128), lambda i: (0, i, 0)),
            pl.BlockSpec((_BN, 128), lambda i: (i, 0)),
            pl.BlockSpec((2, _BN, 128), lambda i: (0, i, 0)),
            pl.BlockSpec((1, 128), lambda i: (0, 0)),
            pl.BlockSpec((128, 128), lambda i: (0, 0)),
            pl.BlockSpec((1024, 128), lambda i: (0, 0)),
        ],
        out_specs=pl.BlockSpec((_BN, 128), lambda i: (i, 0)),
        out_shape=jax.ShapeDtypeStruct((N, 128), F32),
    )(parts, xws, cnt, b, fkbT, fksT)


def _tc_degree_decode(z, dk1bT, dk1sT, dk2b, dk2sv):
    N = z.shape[0]
    nb = N // _BN

    def body(z_ref, b1_ref, s1_ref, b2_ref, s2_ref, o_ref):
        dh = jax.nn.relu(_kan(z_ref[...], b1_ref[...], s1_ref[...]))
        o_ref[...] = jax.nn.relu(_kan_out1(dh, b2_ref[...], s2_ref[...]))

    return pl.pallas_call(
        body,
        grid=(nb,),
        in_specs=[
            pl.BlockSpec((_BN, 64), lambda i: (i, 0)),
            pl.BlockSpec((64, 128), lambda i: (0, 0)),
            pl.BlockSpec((512, 128), lambda i: (0, 0)),
            pl.BlockSpec((1, 128), lambda i: (0, 0)),
            pl.BlockSpec((8, 128), lambda i: (0, 0)),
        ],
        out_specs=pl.BlockSpec((_BN, 1), lambda i: (i, 0)),
        out_shape=jax.ShapeDtypeStruct((N, 1), F32),
    )(z, dk1bT, dk1sT, dk2b, dk2sv)


def _tc_edge_decode(ef, e1bT, e1sT, e2b, e2sv):
    E = ef.shape[0]
    nb = E // _BE

    def body(ef_ref, b1_ref, s1_ref, b2_ref, s2_ref, o_ref):
        t = _kan(ef_ref[...][:, :64], b1_ref[...], s1_ref[...])
        o_ref[...] = jax.nn.sigmoid(_kan_out1(t, b2_ref[...], s2_ref[...]))

    return pl.pallas_call(
        body,
        grid=(nb,),
        in_specs=[
            pl.BlockSpec((_BE, 128), lambda i: (i, 0)),
            pl.BlockSpec((64, 128), lambda i: (0, 0)),
            pl.BlockSpec((512, 128), lambda i: (0, 0)),
            pl.BlockSpec((1, 128), lambda i: (0, 0)),
            pl.BlockSpec((8, 128), lambda i: (0, 0)),
        ],
        out_specs=pl.BlockSpec((_BE, 1), lambda i: (i, 0)),
        out_shape=jax.ShapeDtypeStruct((E, 1), F32),
    )(ef, e1bT, e1sT, e2b, e2sv)


def _sc_degree(dst3, ones_in, zeros16):
    nch, ch = dst3.shape[1], dst3.shape[2]
    stripe = _NP // _NS
    mesh = plsc.VectorSubcoreMesh(core_axis_name="c", subcore_axis_name="s")

    @functools.partial(
        pl.kernel,
        out_type=jax.ShapeDtypeStruct((_NC, _NP, 128), F32),
        mesh=mesh,
        scratch_types=[
            pltpu.VMEM((nch, ch), jnp.int32),
            pltpu.VMEM((ch, 128), F32),
            pltpu.VMEM_SHARED((_NP, 128), F32),
        ],
    )
    def k(dst_h, ones_h, zero_h, out_h, dst_v, ones_v, acc_sh):
        c = lax.axis_index("c")
        s = lax.axis_index("s")
        w = c * _NS + s
        pltpu.sync_copy(dst_h.at[w], dst_v)
        pltpu.sync_copy(ones_h, ones_v)
        pltpu.sync_copy(zero_h, acc_sh.at[pl.ds(s * stripe, stripe)])
        plsc.subcore_barrier()

        def body(i, carry):
            pltpu.sync_copy(ones_v, acc_sh.at[dst_v.at[i]], add=True)
            return carry

        lax.fori_loop(0, nch, body, 0)
        plsc.subcore_barrier()
        pltpu.sync_copy(acc_sh.at[pl.ds(s * stripe, stripe)],
                        out_h.at[c, pl.ds(s * stripe, stripe)])

    return k(dst3, ones_in, zeros16)


def _sc_conv(table, src3, dst3, zeros_stage):
    N, D = table.shape
    nch, ch = src3.shape[1], src3.shape[2]
    stripe = _NP // _NS
    mesh = plsc.VectorSubcoreMesh(core_axis_name="c", subcore_axis_name="s")

    @functools.partial(
        pl.kernel,
        out_type=jax.ShapeDtypeStruct((_NC, _NP, D), F32),
        mesh=mesh,
        scratch_types=[
            pltpu.VMEM((nch, ch), jnp.int32),
            pltpu.VMEM((nch, ch), jnp.int32),
            pltpu.VMEM((ch, D), F32),
            pltpu.VMEM_SHARED((_NP, D), F32),
            pltpu.SemaphoreType.DMA,
        ],
    )
    def k(table_h, src_h, dst_h, zero_h, out_h,
          src_v, dst_v, rows_v, acc_sh, sem):
        c = lax.axis_index("c")
        s = lax.axis_index("s")
        w = c * _NS + s
        pltpu.sync_copy(src_h.at[w], src_v)
        pltpu.sync_copy(dst_h.at[w], dst_v)
        pltpu.sync_copy(zero_h, acc_sh.at[pl.ds(s * stripe, stripe)])
        plsc.subcore_barrier()

        def body(i, carry):
            pltpu.async_copy(table_h.at[src_v.at[i]], rows_v, sem).wait()
            pltpu.sync_copy(rows_v, acc_sh.at[dst_v.at[i]], add=True)
            return carry

        lax.fori_loop(0, nch, body, 0)
        plsc.subcore_barrier()
        pltpu.sync_copy(acc_sh.at[pl.ds(s * stripe, stripe)],
                        out_h.at[c, pl.ds(s * stripe, stripe)])

    return k(table, src3, dst3, zeros_stage)


def _sc_edge_features(z, p03, p13):
    N, D = z.shape
    nch, ch = p03.shape[1], p03.shape[2]
    per_w = nch * ch
    E = _NW * per_w
    mesh = plsc.VectorSubcoreMesh(core_axis_name="c", subcore_axis_name="s")

    @functools.partial(
        pl.kernel,
        out_type=jax.ShapeDtypeStruct((E, D), F32),
        mesh=mesh,
        scratch_types=[
            pltpu.VMEM((nch, ch), jnp.int32),
            pltpu.VMEM((nch, ch), jnp.int32),
            pltpu.VMEM((ch, D), F32),
            pltpu.VMEM((ch, D), F32),
            pltpu.VMEM((ch, D), F32),
            pltpu.SemaphoreType.DMA,
            pltpu.SemaphoreType.DMA,
        ],
    )
    def k(z_h, p0_h, p1_h, out_h, p0_v, p1_v, za_v, zb_v, ef_v, sem0, sem1):
        c = lax.axis_index("c")
        s = lax.axis_index("s")
        w = c * _NS + s
        pltpu.sync_copy(p0_h.at[w], p0_v)
        pltpu.sync_copy(p1_h.at[w], p1_v)

        def body(i, carry):
            cpa = pltpu.async_copy(z_h.at[p0_v.at[i]], za_v, sem0)
            cpb = pltpu.async_copy(z_h.at[p1_v.at[i]], zb_v, sem1)
            cpa.wait()
            cpb.wait()

            def row(r, carry2):
                for j in range(D // 16):
                    sl = pl.ds(j * 16, 16)
                    ef_v[r, sl] = za_v[r, sl] * zb_v[r, sl]
                return carry2

            lax.fori_loop(0, ch, row, 0)
            pltpu.sync_copy(ef_v, out_h.at[pl.ds(w * per_w + i * ch, ch)])
            return carry

        lax.fori_loop(0, nch, body, 0)

    return k(z, p03, p13)


def kernel(x, edge_index, edge_index_for_edge_prediction, W1, b1, W2, b2,
           ek_b, ek_s, Wf, bf, fk_b, fk_s, edk1_b, edk1_s, edk2_b, edk2_s,
           dk1_b, dk1_s, dk2_b, dk2_s):
    N = x.shape[0]
    E = edge_index.shape[1]
    per_w = E // _NW
    ch = 80
    nch = per_w // ch

    src3 = edge_index[0].reshape(_NW, nch, ch)
    dst3 = edge_index[1].reshape(_NW, nch, ch)
    p03 = edge_index_for_edge_prediction[0].reshape(_NW, nch, ch)
    p13 = edge_index_for_edge_prediction[1].reshape(_NW, nch, ch)

    zeros128 = jnp.zeros((_NP // _NS, 128), F32)
    ones128 = jnp.ones((ch, 128), F32)

    ekbT = ek_b.T
    eksT = ek_s.transpose(2, 1, 0).reshape(8 * 128, 64)
    fkbT = fk_b.T
    fksT = fk_s.transpose(2, 1, 0).reshape(8 * 128, 128)
    e1bT = edk1_b.T
    e1sT = edk1_s.transpose(2, 1, 0).reshape(8 * 64, 128)
    e2sv = edk2_s.transpose(2, 1, 0).reshape(8, 128)
    d1bT = dk1_b.T
    d1sT = dk1_s.transpose(2, 1, 0).reshape(8 * 64, 128)
    d2sv = dk2_s.transpose(2, 1, 0).reshape(8, 128)

    b1r = b1.reshape(1, 128)
    b2r = b2.reshape(1, 128)
    bfr = bf.reshape(1, 128)

    cnt = _sc_degree(dst3, ones128, zeros128)

    xws1 = _tc_scale_matmul(x, W1, cnt)
    parts1 = _sc_conv(xws1, src3, dst3, zeros128)
    xws2 = _tc_combine_matmul(parts1, xws1, cnt, b1r, W2)
    parts2 = _sc_conv(xws2, src3, dst3, zeros128)
    z, z128, xws3 = _tc_combine_kan_matmul(parts2, xws2, cnt, b2r, ekbT, eksT,
                                           Wf)
    parts3 = _sc_conv(xws3, src3, dst3, zeros128)
    ef = _sc_edge_features(z128, p03, p13)
    rx = _tc_combine_kan(parts3, xws3, cnt, bfr, fkbT, fksT)
    pd = _tc_degree_decode(z, d1bT, d1sT, dk2_b, d2sv)
    probs = _tc_edge_decode(ef, e1bT, e1sT, edk2_b, e2sv)

    return (rx, probs[:, 0], pd[:, 0], z)
